# baseline (speedup 1.0000x reference)
import functools

import jax
import jax.numpy as jnp
from jax import lax
from jax.experimental import pallas as pl
from jax.experimental.pallas import tpu as pltpu
from jax.experimental.pallas import tpu_sc as plsc


@functools.lru_cache(maxsize=None)
def _build_sc_kernel(B, H, W, C, P):
    info = plsc.get_sparse_core_info()
    NC, NS, L = info.num_cores, info.num_subcores, info.num_lanes
    NW = NC * NS
    R = B * P
    TPB = NW // B
    YPT = H // TPB
    CAP = 64
    CAPG = CAP - L
    assert NW % B == 0 and H % TPB == 0 and YPT % 2 == 0
    assert P % L == 0 and C % L == 0 and C % 8 == 0 and W % 128 == 0
    SCAN_U = 16
    n_scan = P // L // SCAN_U

    mesh = plsc.VectorSubcoreMesh(core_axis_name="c", subcore_axis_name="s")

    @functools.partial(
        pl.kernel,
        mesh=mesh,
        out_type=jax.ShapeDtypeStruct((R + NW * CAP, 128), jnp.float32),
        compiler_params=pltpu.CompilerParams(
            needs_layout_passes=False, use_tc_tiling_on_sc=True
        ),
        scratch_types=[
            pltpu.VMEM((P,), jnp.int32),
            pltpu.VMEM((P,), jnp.int32),
            pltpu.VMEM((P,), jnp.float32),
            pltpu.VMEM((1, C, W), jnp.float32),
            pltpu.VMEM((1, C, W), jnp.float32),
            pltpu.VMEM((L,), jnp.int32),
            pltpu.VMEM((L,), jnp.int32),
            pltpu.VMEM((CAP,), jnp.int32),
            pltpu.VMEM((CAP,), jnp.int32),
            pltpu.VMEM((CAP,), jnp.float32),
            pltpu.VMEM((CAP,), jnp.float32),
            pltpu.VMEM((CAP,), jnp.int32),
            pltpu.VMEM((CAP,), jnp.int32),
            pltpu.VMEM((CAP, 128), jnp.float32),
            pltpu.VMEM((CAP, 128), jnp.float32),
            pltpu.SemaphoreType.DMA,
            pltpu.SemaphoreType.DMA,
            pltpu.SemaphoreType.DMA,
            pltpu.SemaphoreType.DMA,
        ],
    )
    def sc_kernel(emb, yy, xx, mm, out, y_v, x_v, m_v, pla, plb, pi0, pi1,
                  xl0, xl1, ml0, ml1, il0, il1, rw0, rw1,
                  psem0, psem1, ssem0, ssem1):
        wid = lax.axis_index("s") * NC + lax.axis_index("c")
        b = wid // TPB
        ylo = (wid % TPB) * YPT
        pltpu.sync_copy(yy.at[pl.ds(b * P, P)], y_v)
        pltpu.sync_copy(xx.at[pl.ds(b * P, P)], x_v)
        pltpu.sync_copy(mm.at[pl.ds(b * P, P)], m_v)

        iota = lax.iota(jnp.int32, L)
        zero16 = jnp.zeros((L,), jnp.int32)
        dump0 = R + wid * CAP
        row0 = b * H + ylo
        pi0[pl.ds(0, L)] = lax.broadcast(row0, (L,))
        pi1[pl.ds(0, L)] = lax.broadcast(row0 + 1, (L,))
        pltpu.async_copy(emb.at[pi0.at[pl.ds(0, 1)]], pla, psem0)
        pltpu.async_copy(emb.at[pi1.at[pl.ds(0, 1)]], plb, psem1)

        def extract(k, par, plane, pi, il, xl, ml, rows, psem, ssem, n):
            pltpu.make_async_copy(
                emb.at[pi.at[pl.ds(0, 1)]], plane, psem).wait()

            def ext_body(g, carry):
                g16 = g * L
                xv = xl[pl.ds(g16, L)]
                mv = ml[pl.ds(g16, L)]
                for i in range(L):
                    slot = g16 + i

                    @pl.when(slot < n)
                    def _():
                        xs = lax.broadcast(xv[i], (L,))
                        ms = lax.broadcast(mv[i], (L,))
                        for d in range(C // L):
                            v = plsc.load_gather(
                                plane, [zero16, iota + (d * L), xs])
                            rows[slot, pl.ds(d * L, L)] = v * ms
                return carry

            lax.fori_loop(0, (n + L - 1) // L, ext_body, 0)
            pltpu.async_copy(rows, out.at[il], ssem)

            @pl.when(2 * k + par + 2 < YPT)
            def _():
                pi[pl.ds(0, L)] = lax.broadcast(row0 + 2 * k + par + 2, (L,))
                pltpu.async_copy(emb.at[pi.at[pl.ds(0, 1)]], plane, psem)

        def step(k, carry):
            yg0 = ylo + 2 * k
            @pl.when(k > 0)
            def _():
                pltpu.make_async_copy(
                    rw0, out.at[pl.ds(0, CAP)], ssem0).wait()
                pltpu.make_async_copy(
                    rw1, out.at[pl.ds(0, CAP)], ssem1).wait()
            for j in range(CAP // L):
                il0[pl.ds(j * L, L)] = dump0 + (j * L) + iota
                il1[pl.ds(j * L, L)] = dump0 + (j * L) + iota

            def scan_body(v, carry):
                c0v, c1v = carry
                for u in range(SCAN_U):
                    off = v * (SCAN_U * L) + u * L
                    yv = y_v[pl.ds(off, L)]
                    xv = x_v[pl.ds(off, L)]
                    mv = m_v[pl.ds(off, L)]
                    pv = b * P + off + iota
                    m0 = yv == yg0
                    m1 = yv == (yg0 + 1)
                    n0 = plsc.all_reduce_population_count(m0)
                    n1 = plsc.all_reduce_population_count(m1)
                    c0 = jnp.minimum(c0v, CAPG)[0]
                    c1 = jnp.minimum(c1v, CAPG)[0]
                    plsc.store_compressed(xl0.at[pl.ds(c0, L)], xv, mask=m0)
                    plsc.store_compressed(ml0.at[pl.ds(c0, L)], mv, mask=m0)
                    plsc.store_compressed(il0.at[pl.ds(c0, L)], pv, mask=m0)
                    plsc.store_compressed(xl1.at[pl.ds(c1, L)], xv, mask=m1)
                    plsc.store_compressed(ml1.at[pl.ds(c1, L)], mv, mask=m1)
                    plsc.store_compressed(il1.at[pl.ds(c1, L)], pv, mask=m1)
                    c0v = c0v + n0
                    c1v = c1v + n1
                return (c0v, c1v)

            zero = jnp.zeros((L,), jnp.int32)
            c0v, c1v = lax.fori_loop(0, n_scan, scan_body, (zero, zero))
            n0 = jnp.minimum(c0v[0], CAPG)
            n1 = jnp.minimum(c1v[0], CAPG)

            extract(k, 0, pla, pi0, il0, xl0, ml0, rw0, psem0, ssem0, n0)
            extract(k, 1, plb, pi1, il1, xl1, ml1, rw1, psem1, ssem1, n1)
            return carry

        lax.fori_loop(0, YPT // 2, step, 0)
        pltpu.make_async_copy(rw0, out.at[pl.ds(0, CAP)], ssem0).wait()
        pltpu.make_async_copy(rw1, out.at[pl.ds(0, CAP)], ssem1).wait()

    return sc_kernel


def kernel(embeddings, coords, mask):
    B, H, W, C = embeddings.shape
    P = coords.shape[1]
    emb_s = jnp.transpose(embeddings, (0, 1, 3, 2)).reshape(B * H, C, W)
    c32 = coords.astype(jnp.int32)
    yy = c32[..., 0].reshape(-1)
    xx = c32[..., 1].reshape(-1)
    mm = mask.reshape(-1)
    out = _build_sc_kernel(B, H, W, C, P)(emb_s, yy, xx, mm)
    return out[:B * P, :C].reshape(B, P, C)


# speedup vs baseline: 1.0031x; 1.0031x over previous
import functools

import jax
import jax.numpy as jnp
from jax import lax
from jax.experimental import pallas as pl
from jax.experimental.pallas import tpu as pltpu
from jax.experimental.pallas import tpu_sc as plsc


@functools.lru_cache(maxsize=None)
def _build_sc_kernel(B, H, W, C, P):
    info = plsc.get_sparse_core_info()
    NC, NS, L = info.num_cores, info.num_subcores, info.num_lanes
    NW = NC * NS
    R = B * P
    TPB = NW // B
    YPT = H // TPB
    CAP = 64
    CAPG = CAP - L
    assert NW % B == 0 and H % TPB == 0 and YPT % 2 == 0
    assert P % L == 0 and C % L == 0 and C % 8 == 0 and W % 128 == 0
    SCAN_U = 16
    n_scan = P // L // SCAN_U

    mesh = plsc.VectorSubcoreMesh(core_axis_name="c", subcore_axis_name="s")

    @functools.partial(
        pl.kernel,
        mesh=mesh,
        out_type=jax.ShapeDtypeStruct((R + NW * CAP, 128), jnp.float32),
        compiler_params=pltpu.CompilerParams(
            needs_layout_passes=False, use_tc_tiling_on_sc=True
        ),
        scratch_types=[
            pltpu.VMEM((P,), jnp.int32),
            pltpu.VMEM((P,), jnp.int32),
            pltpu.VMEM((P,), jnp.float32),
            pltpu.VMEM((C, W), jnp.float32),
            pltpu.VMEM((C, W), jnp.float32),
            pltpu.VMEM((CAP,), jnp.int32),
            pltpu.VMEM((CAP,), jnp.int32),
            pltpu.VMEM((CAP,), jnp.float32),
            pltpu.VMEM((CAP,), jnp.float32),
            pltpu.VMEM((CAP,), jnp.int32),
            pltpu.VMEM((CAP,), jnp.int32),
            pltpu.VMEM((CAP, 128), jnp.float32),
            pltpu.VMEM((CAP, 128), jnp.float32),
            pltpu.SemaphoreType.DMA,
            pltpu.SemaphoreType.DMA,
            pltpu.SemaphoreType.DMA,
            pltpu.SemaphoreType.DMA,
        ],
    )
    def sc_kernel(emb, yy, xx, mm, out, y_v, x_v, m_v, pla, plb,
                  xl0, xl1, ml0, ml1, il0, il1, rw0, rw1,
                  psem0, psem1, ssem0, ssem1):
        wid = lax.axis_index("s") * NC + lax.axis_index("c")
        b = wid // TPB
        ylo = (wid % TPB) * YPT
        pltpu.sync_copy(yy.at[pl.ds(b * P, P)], y_v)
        pltpu.sync_copy(xx.at[pl.ds(b * P, P)], x_v)
        pltpu.sync_copy(mm.at[pl.ds(b * P, P)], m_v)

        iota = lax.iota(jnp.int32, L)
        dump0 = R + wid * CAP
        row0 = b * H + ylo
        pltpu.async_copy(emb.at[row0], pla, psem0)
        pltpu.async_copy(emb.at[row0 + 1], plb, psem1)

        def extract(k, par, plane, il, xl, ml, rows, psem, ssem, n):
            pltpu.make_async_copy(emb.at[row0], plane, psem).wait()

            def ext_body(g, carry):
                g16 = g * L
                xv = xl[pl.ds(g16, L)]
                mv = ml[pl.ds(g16, L)]
                for i in range(L):
                    slot = g16 + i

                    @pl.when(slot < n)
                    def _():
                        xs = lax.broadcast(xv[i], (L,))
                        ms = lax.broadcast(mv[i], (L,))
                        for d in range(C // L):
                            v = plsc.load_gather(
                                plane, [iota + (d * L), xs])
                            rows[slot, pl.ds(d * L, L)] = v * ms
                return carry

            lax.fori_loop(0, (n + L - 1) // L, ext_body, 0)
            pltpu.async_copy(rows, out.at[il], ssem)

            @pl.when(2 * k + par + 2 < YPT)
            def _():
                pltpu.async_copy(emb.at[row0 + 2 * k + par + 2], plane, psem)

        def step(k, carry):
            yg0 = ylo + 2 * k
            @pl.when(k > 0)
            def _():
                pltpu.make_async_copy(
                    rw0, out.at[pl.ds(0, CAP)], ssem0).wait()
                pltpu.make_async_copy(
                    rw1, out.at[pl.ds(0, CAP)], ssem1).wait()
            for j in range(CAP // L):
                il0[pl.ds(j * L, L)] = dump0 + (j * L) + iota
                il1[pl.ds(j * L, L)] = dump0 + (j * L) + iota

            def scan_body(v, carry):
                c0v, c1v = carry
                for u in range(SCAN_U):
                    off = v * (SCAN_U * L) + u * L
                    yv = y_v[pl.ds(off, L)]
                    xv = x_v[pl.ds(off, L)]
                    mv = m_v[pl.ds(off, L)]
                    pv = b * P + off + iota
                    m0 = yv == yg0
                    m1 = yv == (yg0 + 1)
                    n0 = plsc.all_reduce_population_count(m0)
                    n1 = plsc.all_reduce_population_count(m1)
                    c0 = jnp.minimum(c0v, CAPG)[0]
                    c1 = jnp.minimum(c1v, CAPG)[0]
                    plsc.store_compressed(xl0.at[pl.ds(c0, L)], xv, mask=m0)
                    plsc.store_compressed(ml0.at[pl.ds(c0, L)], mv, mask=m0)
                    plsc.store_compressed(il0.at[pl.ds(c0, L)], pv, mask=m0)
                    plsc.store_compressed(xl1.at[pl.ds(c1, L)], xv, mask=m1)
                    plsc.store_compressed(ml1.at[pl.ds(c1, L)], mv, mask=m1)
                    plsc.store_compressed(il1.at[pl.ds(c1, L)], pv, mask=m1)
                    c0v = c0v + n0
                    c1v = c1v + n1
                return (c0v, c1v)

            zero = jnp.zeros((L,), jnp.int32)
            c0v, c1v = lax.fori_loop(0, n_scan, scan_body, (zero, zero))
            n0 = jnp.minimum(c0v[0], CAPG)
            n1 = jnp.minimum(c1v[0], CAPG)

            extract(k, 0, pla, il0, xl0, ml0, rw0, psem0, ssem0, n0)
            extract(k, 1, plb, il1, xl1, ml1, rw1, psem1, ssem1, n1)
            return carry

        lax.fori_loop(0, YPT // 2, step, 0)
        pltpu.make_async_copy(rw0, out.at[pl.ds(0, CAP)], ssem0).wait()
        pltpu.make_async_copy(rw1, out.at[pl.ds(0, CAP)], ssem1).wait()

    return sc_kernel


def kernel(embeddings, coords, mask):
    B, H, W, C = embeddings.shape
    P = coords.shape[1]
    emb_s = jnp.transpose(embeddings, (0, 1, 3, 2)).reshape(B * H, C, W)
    c32 = coords.astype(jnp.int32)
    yy = c32[..., 0].reshape(-1)
    xx = c32[..., 1].reshape(-1)
    mm = mask.reshape(-1)
    out = _build_sc_kernel(B, H, W, C, P)(emb_s, yy, xx, mm)
    return out[:B * P, :C].reshape(B, P, C)
